# baseline (speedup 1.0000x reference)
import functools

import jax
import jax.numpy as jnp
from jax import lax
from jax.experimental import pallas as pl
from jax.experimental.pallas import tpu as pltpu
from jax.experimental.pallas import tpu_sc as plsc

_N = 100000
_E = 1600000
_H = 32
_EPS = 1e-08
_MAX_DELTA_LOG = 0.3

_NC = 2
_NS = 16
_NW = _NC * _NS
_TPN = 3136
_NP = _NW * _TPN
_ROWS = _NP // 128
_BLK = 4096
_NV = _BLK // 16
_NTS = 8
_RD = _TPN + 16
_RSTART_MAX = ((_N + 1 - _RD) // 8) * 8


def _layer_body(hu_ref, rawseg_ref, deg_ref, w_ref, nei_ref, self_ref):
    x = hu_ref[...]
    s = jnp.sum(x)
    ss = jnp.sum(x * x)
    mu = s / _N
    var = (ss - s * s / _N) / (_N - 1)
    sigma = jnp.sqrt(var) + _EPS
    inv = 1.0 / sigma
    h0 = (x - mu) * inv
    sg = (rawseg_ref[...] - deg_ref[...] * mu) * inv
    accn = jnp.zeros_like(x)
    accs = jnp.zeros_like(x)
    for h in range(_H):
        wn1 = w_ref[0, h]
        ws1 = w_ref[1, h]
        bb = w_ref[2, h]
        wn2 = w_ref[3, h]
        ws2 = w_ref[4, h]
        h1 = jnp.maximum(bb + sg * wn1 + h0 * ws1, 0.0)
        accn = accn + h1 * wn2
        accs = accs + h1 * ws2
    nei_ref[...] = accn
    self_ref[...] = accs + w_ref[5, 0]


def _seg_body(final, table_hbm, idx_hbm, offs_hbm, sself_hbm, *rest):
    if final:
        (out_hbm, table_v, offs_v, idxbuf_v, cumvec_v, lvp_v, barr_v, seg_v,
         sself_v, deg_v, o0_v, dsem, tsem) = rest
        deg_hbm = None
    else:
        (out_hbm, deg_hbm, table_v, offs_v, idxbuf_v, cumvec_v, lvp_v, barr_v,
         seg_v, sself_v, deg_v, o0_v, dsem, tsem) = rest

    wid = lax.axis_index("s") * _NC + lax.axis_index("c")
    r0 = pl.multiple_of(wid * _TPN, 8)
    rstart = pl.multiple_of(jnp.minimum(r0, jnp.int32(_RSTART_MAX)), 8)

    def gi(x):
        return jnp.minimum(r0 + x, _N) - rstart

    pltpu.sync_copy(offs_hbm.at[pl.ds(0, 16)], o0_v)
    pltpu.sync_copy(offs_hbm.at[pl.ds(rstart, _RD)], offs_v.at[pl.ds(0, _RD)])
    if final:
        pltpu.sync_copy(sself_hbm.at[pl.ds(r0, _TPN)], sself_v)

    tot = table_hbm.shape[0]
    ch = ((tot // _NTS) // 8) * 8
    bnds = [(c * ch, min((c + 1) * ch, tot)) for c in range(_NTS - 1)]
    bnds.append(((_NTS - 1) * ch, tot))
    for a, b in bnds:
        pltpu.async_copy(table_hbm.at[pl.ds(a, b - a)],
                         table_v.at[pl.ds(a, b - a)], tsem)

    iota = lax.iota(jnp.int32, 16)
    o0 = o0_v[pl.ds(0, 16)][0]
    sh = r0 - rstart
    s_start = offs_v[pl.ds(sh, 16)][0] - o0
    s_end = offs_v[pl.ds(gi(_TPN), 16)][0] - o0
    a0 = lax.bitwise_and(s_start, jnp.int32(-8))
    nblk = (s_end - a0) // _BLK + 1

    w00 = pl.multiple_of(jnp.minimum(a0, jnp.int32(_E - _BLK)), 8)
    pltpu.async_copy(idx_hbm.at[pl.ds(w00, _BLK)],
                     idxbuf_v.at[pl.ds(0, _BLK)], dsem)

    with jax.named_scope("sc_stage_in"):
        for a, b in bnds:
            pltpu.make_async_copy(table_hbm.at[pl.ds(a, b - a)],
                                  table_v.at[pl.ds(a, b - a)], tsem).wait()

    def block_body(k, carry):
        nr, tp = carry
        b0 = a0 + k * _BLK
        off = pl.multiple_of(lax.bitwise_and(k, 1) * _BLK, 8)
        pltpu.make_async_copy(idx_hbm.at[pl.ds(0, _BLK)],
                              idxbuf_v.at[pl.ds(off, _BLK)], dsem).wait()

        @pl.when(k + 1 < nblk)
        def _prefetch():
            w0n = pl.multiple_of(
                jnp.minimum(b0 + _BLK, jnp.int32(_E - _BLK)), 8)
            offn = pl.multiple_of(lax.bitwise_and(k + 1, 1) * _BLK, 8)
            pltpu.async_copy(idx_hbm.at[pl.ds(w0n, _BLK)],
                             idxbuf_v.at[pl.ds(offn, _BLK)], dsem)

        interior = (b0 >= s_start) & (b0 + _BLK <= s_end)

        @pl.when(interior)
        def _fast():
            @plsc.parallel_loop(0, _NV, 1, unroll=8)
            def pa(v):
                bi = off + v * 16 + iota
                nid = plsc.load_gather(idxbuf_v, [bi])
                g = plsc.load_gather(table_v, [nid])
                cumvec_v[pl.ds(v * 16, 16)] = plsc.cumsum(g)

        @pl.when(jnp.logical_not(interior))
        def _slow():
            w0 = pl.multiple_of(jnp.minimum(b0, jnp.int32(_E - _BLK)), 8)

            @plsc.parallel_loop(0, _NV, 1, unroll=4)
            def pa(v):
                jg = b0 + v * 16 + iota
                m = (jg >= s_start) & (jg < s_end)
                bi = jnp.minimum(jg - w0, _BLK - 1) + off
                nid = plsc.load_gather(idxbuf_v, [bi])
                g = plsc.load_gather(table_v, [nid])
                g = jnp.where(m, g, jnp.float32(0.0))
                cumvec_v[pl.ds(v * 16, 16)] = plsc.cumsum(g)

        @plsc.parallel_loop(0, _NV // 16, 1, unroll=2)
        def pa2(u):
            idxs = (u * 16 + iota) * 16 + 15
            svals = plsc.load_gather(cumvec_v, [idxs])
            lvp_v[pl.ds(u * 16, 16)] = plsc.cumsum(svals)

        gt = plsc.load_gather(lvp_v, [iota * 16 + 15])
        cum = plsc.cumsum(gt)
        for g in range(1, _NV // 16):
            lvp_v[pl.ds(g * 16, 16)] = lvp_v[pl.ds(g * 16, 16)] + cum[g - 1]
        blk_total = cum[_NV // 16 - 1]

        b1 = b0 + _BLK

        def gcond(c):
            return c[1]

        def gbody(c):
            nrg, _ = c
            qr = nrg + iota * 16 + 15
            p1 = plsc.load_gather(offs_v, [gi(qr)]) - o0
            c1m = (p1 < b1) & (qr <= _TPN)
            cnt1 = plsc.all_reduce_population_count(c1m)[0]
            return (nrg + cnt1 * 16, cnt1 >= 16)

        nr_c, _ = lax.while_loop(gcond, gbody, (nr, jnp.bool_(True)))
        q2 = nr_c + iota
        p2 = plsc.load_gather(offs_v, [gi(q2)]) - o0
        c2m = (p2 < b1) & (q2 <= _TPN)
        nr_end = nr_c + plsc.all_reduce_population_count(c2m)[0]
        nch = (nr_end - nr + 15) >> 4

        @plsc.parallel_loop(0, nch, 1, unroll=2)
        def pb(c):
            rvec = nr + c * 16 + iota
            mask = rvec < nr_end
            p = plsc.load_gather(offs_v, [gi(jnp.minimum(rvec, _TPN))]) - o0
            sl = p - b0
            vv = lax.shift_right_logical(sl, 4)
            ll = lax.bitwise_and(sl, 15)
            lvpexc = jnp.where(
                vv > 0,
                plsc.load_gather(lvp_v, [jnp.clip(vv - 1, 0, _NV - 1)]),
                jnp.float32(0.0))
            intra = jnp.where(
                ll > 0,
                plsc.load_gather(cumvec_v, [jnp.clip(sl - 1, 0, _BLK - 1)]),
                jnp.float32(0.0))
            plsc.store_scatter(barr_v, [rvec], tp + lvpexc + intra, mask=mask)

        return (nr_end, tp + blk_total)

    with jax.named_scope("sc_blocks"):
        lax.fori_loop(0, nblk, block_body, (jnp.int32(0), jnp.float32(0.0)))

    @plsc.parallel_loop(0, _TPN // 16, 1, unroll=8)
    def segv(v):
        a = plsc.load_gather(barr_v, [v * 16 + iota])
        b = plsc.load_gather(barr_v, [v * 16 + 1 + iota])
        seg = b - a
        if final:
            z = seg + sself_v[pl.ds(v * 16, 16)]
            e = jnp.exp(z + z)
            seg = _MAX_DELTA_LOG * (1.0 - 2.0 / (e + 1.0))
        else:
            oa = plsc.load_gather(offs_v, [gi(v * 16 + iota)])
            ob = plsc.load_gather(offs_v, [gi(v * 16 + 1 + iota)])
            deg_v[pl.ds(v * 16, 16)] = (ob - oa).astype(jnp.float32)
        seg_v[pl.ds(v * 16, 16)] = seg
    pltpu.sync_copy(seg_v, out_hbm.at[pl.ds(r0, _TPN)])
    if not final:
        pltpu.sync_copy(deg_v, deg_hbm.at[pl.ds(r0, _TPN)])


def _seg_sum(table, idx, offs, sself, final):
    mesh = plsc.VectorSubcoreMesh(core_axis_name="c", subcore_axis_name="s",
                                  num_cores=_NC, num_subcores=_NS)
    out_t = jax.ShapeDtypeStruct((_NP,), jnp.float32)
    fn = pl.kernel(
        functools.partial(_seg_body, final),
        out_type=out_t if final else (out_t, out_t),
        mesh=mesh,
        scratch_types=[
            pltpu.VMEM((_NP,), jnp.float32),
            pltpu.VMEM((_RD + 16,), jnp.int32),
            pltpu.VMEM((2 * _BLK,), jnp.int32),
            pltpu.VMEM((_BLK,), jnp.float32),
            pltpu.VMEM((_NV,), jnp.float32),
            pltpu.VMEM((_TPN + 16,), jnp.float32),
            pltpu.VMEM((_TPN,), jnp.float32),
            pltpu.VMEM((_TPN,), jnp.float32),
            pltpu.VMEM((_TPN,), jnp.float32),
            pltpu.VMEM((16,), jnp.int32),
            pltpu.SemaphoreType.DMA,
            pltpu.SemaphoreType.DMA,
        ],
        compiler_params=pltpu.CompilerParams(needs_layout_passes=False),
    )
    return fn(table, idx, offs, sself)


def kernel(hu_scalar, neighbor_indices, neighbor_offsets,
           W_nei1, W_self1, b1, W_nei2, W_self2, b2):
    hu = hu_scalar.astype(jnp.float32)
    idx = neighbor_indices.astype(jnp.int32)
    offs = neighbor_offsets.astype(jnp.int32)
    hu2 = jnp.pad(hu, (0, _NP - _N)).reshape(_ROWS, 128)

    wpack = jnp.zeros((8, 128), jnp.float32)
    wpack = wpack.at[0, :_H].set(W_nei1.reshape(_H).astype(jnp.float32))
    wpack = wpack.at[1, :_H].set(W_self1.reshape(_H).astype(jnp.float32))
    wpack = wpack.at[2, :_H].set(b1.astype(jnp.float32))
    wpack = wpack.at[3, :_H].set(W_nei2.astype(jnp.float32))
    wpack = wpack.at[4, :_H].set(W_self2.astype(jnp.float32))
    wpack = wpack.at[5, 0].set(b2.reshape(())[...].astype(jnp.float32))

    f32_2d = jax.ShapeDtypeStruct((_ROWS, 128), jnp.float32)
    rawseg0, degf = _seg_sum(hu, idx, offs, hu, final=False)
    s_nei, s_self = pl.pallas_call(
        _layer_body,
        out_shape=(f32_2d, f32_2d),
        in_specs=[pl.BlockSpec((_ROWS, 128), lambda: (0, 0))] * 3
        + [pl.BlockSpec(memory_space=pltpu.SMEM)],
    )(hu2, rawseg0.reshape(_ROWS, 128), degf.reshape(_ROWS, 128), wpack)
    out = _seg_sum(s_nei.reshape(_NP), idx, offs,
                   s_self.reshape(_NP), final=True)
    return out[:_N]


# speedup vs baseline: 4562.9741x; 4562.9741x over previous
import functools

import jax
import jax.numpy as jnp
from jax import lax
from jax.experimental import pallas as pl
from jax.experimental.pallas import tpu as pltpu
from jax.experimental.pallas import tpu_sc as plsc

_N = 100000
_E = 1600000
_H = 32
_EPS = 1e-08
_MAX_DELTA_LOG = 0.3

_NC = 2
_NS = 16
_NW = _NC * _NS
_TPN = 3136
_NP = _NW * _TPN
_ROWS = _NP // 128
_BLK = 4096
_NV = _BLK // 16
_NTS = 8
_RD = _TPN + 16
_OPAD = _N + 8
_RSTART_MAX = _OPAD - _RD


def _layer_body(hu_ref, rawseg_ref, deg_ref, w_ref, nei_ref, self_ref):
    x = hu_ref[...]
    s = jnp.sum(x)
    ss = jnp.sum(x * x)
    mu = s / _N
    var = (ss - s * s / _N) / (_N - 1)
    sigma = jnp.sqrt(var) + _EPS
    inv = 1.0 / sigma
    h0 = (x - mu) * inv
    sg = (rawseg_ref[...] - deg_ref[...] * mu) * inv
    accn = jnp.zeros_like(x)
    accs = jnp.zeros_like(x)
    for h in range(_H):
        wn1 = w_ref[0, h]
        ws1 = w_ref[1, h]
        bb = w_ref[2, h]
        wn2 = w_ref[3, h]
        ws2 = w_ref[4, h]
        h1 = jnp.maximum(bb + sg * wn1 + h0 * ws1, 0.0)
        accn = accn + h1 * wn2
        accs = accs + h1 * ws2
    nei_ref[...] = accn
    self_ref[...] = accs + w_ref[5, 0]


def _seg_body(final, table_hbm, idx_hbm, offs_hbm, sself_hbm, *rest):
    if final:
        (out_hbm, table_v, offs_v, idxbuf_v, cumvec_v, lvp_v, barr_v, seg_v,
         sself_v, deg_v, o0_v, dsem, tsem) = rest
        deg_hbm = None
    else:
        (out_hbm, deg_hbm, table_v, offs_v, idxbuf_v, cumvec_v, lvp_v, barr_v,
         seg_v, sself_v, deg_v, o0_v, dsem, tsem) = rest

    wid = lax.axis_index("s") * _NC + lax.axis_index("c")
    r0 = pl.multiple_of(wid * _TPN, 8)
    rstart = pl.multiple_of(jnp.minimum(r0, jnp.int32(_RSTART_MAX)), 8)

    def gi(x):
        return jnp.minimum(r0 + x, _N) - rstart

    pltpu.sync_copy(offs_hbm.at[pl.ds(0, 16)], o0_v)
    pltpu.sync_copy(offs_hbm.at[pl.ds(rstart, _RD)], offs_v.at[pl.ds(0, _RD)])
    if final:
        pltpu.sync_copy(sself_hbm.at[pl.ds(r0, _TPN)], sself_v)

    tot = table_hbm.shape[0]
    ch = ((tot // _NTS) // 8) * 8
    bnds = [(c * ch, min((c + 1) * ch, tot)) for c in range(_NTS - 1)]
    bnds.append(((_NTS - 1) * ch, tot))
    for a, b in bnds:
        pltpu.async_copy(table_hbm.at[pl.ds(a, b - a)],
                         table_v.at[pl.ds(a, b - a)], tsem)

    iota = lax.iota(jnp.int32, 16)
    o0 = o0_v[pl.ds(0, 16)][0]
    sh = r0 - rstart
    s_start = offs_v[pl.ds(sh, 16)][0] - o0
    s_end = offs_v[pl.ds(gi(_TPN), 16)][0] - o0
    a0 = lax.bitwise_and(s_start, jnp.int32(-8))
    nblk = (s_end - a0) // _BLK + 1

    w00 = pl.multiple_of(jnp.minimum(a0, jnp.int32(_E - _BLK)), 8)
    pltpu.async_copy(idx_hbm.at[pl.ds(w00, _BLK)],
                     idxbuf_v.at[pl.ds(0, _BLK)], dsem)

    with jax.named_scope("sc_stage_in"):
        for a, b in bnds:
            pltpu.make_async_copy(table_hbm.at[pl.ds(a, b - a)],
                                  table_v.at[pl.ds(a, b - a)], tsem).wait()

    def block_body(k, carry):
        nr, tp = carry
        b0 = a0 + k * _BLK
        off = pl.multiple_of(lax.bitwise_and(k, 1) * _BLK, 8)
        pltpu.make_async_copy(idx_hbm.at[pl.ds(0, _BLK)],
                              idxbuf_v.at[pl.ds(off, _BLK)], dsem).wait()

        @pl.when(k + 1 < nblk)
        def _prefetch():
            w0n = pl.multiple_of(
                jnp.minimum(b0 + _BLK, jnp.int32(_E - _BLK)), 8)
            offn = pl.multiple_of(lax.bitwise_and(k + 1, 1) * _BLK, 8)
            pltpu.async_copy(idx_hbm.at[pl.ds(w0n, _BLK)],
                             idxbuf_v.at[pl.ds(offn, _BLK)], dsem)

        interior = (b0 >= s_start) & (b0 + _BLK <= s_end)

        @pl.when(interior)
        def _fast():
            @plsc.parallel_loop(0, _NV, 1, unroll=8)
            def pa(v):
                bi = off + v * 16 + iota
                nid = plsc.load_gather(idxbuf_v, [bi])
                g = plsc.load_gather(table_v, [nid])
                cumvec_v[pl.ds(v * 16, 16)] = plsc.cumsum(g)

        @pl.when(jnp.logical_not(interior))
        def _slow():
            w0 = pl.multiple_of(jnp.minimum(b0, jnp.int32(_E - _BLK)), 8)

            @plsc.parallel_loop(0, _NV, 1, unroll=4)
            def pa(v):
                jg = b0 + v * 16 + iota
                m = (jg >= s_start) & (jg < s_end)
                bi = jnp.minimum(jg - w0, _BLK - 1) + off
                nid = plsc.load_gather(idxbuf_v, [bi])
                g = plsc.load_gather(table_v, [nid])
                g = jnp.where(m, g, jnp.float32(0.0))
                cumvec_v[pl.ds(v * 16, 16)] = plsc.cumsum(g)

        @plsc.parallel_loop(0, _NV // 16, 1, unroll=2)
        def pa2(u):
            idxs = (u * 16 + iota) * 16 + 15
            svals = plsc.load_gather(cumvec_v, [idxs])
            lvp_v[pl.ds(u * 16, 16)] = plsc.cumsum(svals)

        gt = plsc.load_gather(lvp_v, [iota * 16 + 15])
        cum = plsc.cumsum(gt)
        for g in range(1, _NV // 16):
            lvp_v[pl.ds(g * 16, 16)] = lvp_v[pl.ds(g * 16, 16)] + cum[g - 1]
        blk_total = cum[_NV // 16 - 1]

        b1 = b0 + _BLK

        def gcond(c):
            return c[1]

        def gbody(c):
            nrg, _ = c
            qr = nrg + iota * 16 + 15
            p1 = plsc.load_gather(offs_v, [gi(qr)]) - o0
            c1m = (p1 < b1) & (qr <= _TPN)
            cnt1 = plsc.all_reduce_population_count(c1m)[0]
            return (nrg + cnt1 * 16, cnt1 >= 16)

        nr_c, _ = lax.while_loop(gcond, gbody, (nr, jnp.bool_(True)))
        q2 = nr_c + iota
        p2 = plsc.load_gather(offs_v, [gi(q2)]) - o0
        c2m = (p2 < b1) & (q2 <= _TPN)
        nr_end = nr_c + plsc.all_reduce_population_count(c2m)[0]
        nch = (nr_end - nr + 15) >> 4

        @plsc.parallel_loop(0, nch, 1, unroll=2)
        def pb(c):
            rvec = nr + c * 16 + iota
            mask = rvec < nr_end
            p = plsc.load_gather(offs_v, [gi(jnp.minimum(rvec, _TPN))]) - o0
            sl = p - b0
            vv = lax.shift_right_logical(sl, 4)
            ll = lax.bitwise_and(sl, 15)
            lvpexc = jnp.where(
                vv > 0,
                plsc.load_gather(lvp_v, [jnp.clip(vv - 1, 0, _NV - 1)]),
                jnp.float32(0.0))
            intra = jnp.where(
                ll > 0,
                plsc.load_gather(cumvec_v, [jnp.clip(sl - 1, 0, _BLK - 1)]),
                jnp.float32(0.0))
            plsc.store_scatter(barr_v, [rvec], tp + lvpexc + intra, mask=mask)

        return (nr_end, tp + blk_total)

    with jax.named_scope("sc_blocks"):
        lax.fori_loop(0, nblk, block_body, (jnp.int32(0), jnp.float32(0.0)))

    @plsc.parallel_loop(0, _TPN // 16, 1, unroll=8)
    def segv(v):
        a = plsc.load_gather(barr_v, [v * 16 + iota])
        b = plsc.load_gather(barr_v, [v * 16 + 1 + iota])
        seg = b - a
        if final:
            z = seg + sself_v[pl.ds(v * 16, 16)]
            e = jnp.exp(z + z)
            seg = _MAX_DELTA_LOG * (1.0 - 2.0 / (e + 1.0))
        else:
            oa = plsc.load_gather(offs_v, [gi(v * 16 + iota)])
            ob = plsc.load_gather(offs_v, [gi(v * 16 + 1 + iota)])
            deg_v[pl.ds(v * 16, 16)] = (ob - oa).astype(jnp.float32)
        seg_v[pl.ds(v * 16, 16)] = seg
    pltpu.sync_copy(seg_v, out_hbm.at[pl.ds(r0, _TPN)])
    if not final:
        pltpu.sync_copy(deg_v, deg_hbm.at[pl.ds(r0, _TPN)])


def _seg_sum(table, idx, offs, sself, final):
    mesh = plsc.VectorSubcoreMesh(core_axis_name="c", subcore_axis_name="s",
                                  num_cores=_NC, num_subcores=_NS)
    out_t = jax.ShapeDtypeStruct((_NP,), jnp.float32)
    fn = pl.kernel(
        functools.partial(_seg_body, final),
        out_type=out_t if final else (out_t, out_t),
        mesh=mesh,
        scratch_types=[
            pltpu.VMEM((_NP,), jnp.float32),
            pltpu.VMEM((_RD + 16,), jnp.int32),
            pltpu.VMEM((2 * _BLK,), jnp.int32),
            pltpu.VMEM((_BLK,), jnp.float32),
            pltpu.VMEM((_NV,), jnp.float32),
            pltpu.VMEM((_TPN + 16,), jnp.float32),
            pltpu.VMEM((_TPN,), jnp.float32),
            pltpu.VMEM((_TPN,), jnp.float32),
            pltpu.VMEM((_TPN,), jnp.float32),
            pltpu.VMEM((16,), jnp.int32),
            pltpu.SemaphoreType.DMA,
            pltpu.SemaphoreType.DMA,
        ],
        compiler_params=pltpu.CompilerParams(needs_layout_passes=False),
    )
    return fn(table, idx, offs, sself)


def kernel(hu_scalar, neighbor_indices, neighbor_offsets,
           W_nei1, W_self1, b1, W_nei2, W_self2, b2):
    hu = hu_scalar.astype(jnp.float32)
    idx = neighbor_indices.astype(jnp.int32)
    offs = jnp.pad(neighbor_offsets.astype(jnp.int32), (0, 7), mode='edge')
    hu2 = jnp.pad(hu, (0, _NP - _N)).reshape(_ROWS, 128)

    wpack = jnp.zeros((8, 128), jnp.float32)
    wpack = wpack.at[0, :_H].set(W_nei1.reshape(_H).astype(jnp.float32))
    wpack = wpack.at[1, :_H].set(W_self1.reshape(_H).astype(jnp.float32))
    wpack = wpack.at[2, :_H].set(b1.astype(jnp.float32))
    wpack = wpack.at[3, :_H].set(W_nei2.astype(jnp.float32))
    wpack = wpack.at[4, :_H].set(W_self2.astype(jnp.float32))
    wpack = wpack.at[5, 0].set(b2.reshape(())[...].astype(jnp.float32))

    f32_2d = jax.ShapeDtypeStruct((_ROWS, 128), jnp.float32)
    rawseg0, degf = _seg_sum(hu, idx, offs, hu, final=False)
    s_nei, s_self = pl.pallas_call(
        _layer_body,
        out_shape=(f32_2d, f32_2d),
        in_specs=[pl.BlockSpec((_ROWS, 128), lambda: (0, 0))] * 3
        + [pl.BlockSpec(memory_space=pltpu.SMEM)],
    )(hu2, rawseg0.reshape(_ROWS, 128), degf.reshape(_ROWS, 128), wpack)
    out = _seg_sum(s_nei.reshape(_NP), idx, offs,
                   s_self.reshape(_NP), final=True)
    return out[:_N]
